# Initial kernel scaffold; baseline (speedup 1.0000x reference)
import functools

import jax
import jax.numpy as jnp
from jax import lax
from jax.experimental import pallas as pl
from jax.experimental.pallas import tpu as pltpu
from jax.experimental.pallas import tpu_sc as plsc

N_NODES = 10000
N_EDGES = 320000
D = 128

NC = 2
NS = 16
NW = NC * NS
G = 128
GP = -(-N_EDGES // (NW * G))
E_PAD = NW * GP * G
M = 10240
RT = M // NS
RB = RT // G

_sc_mesh = plsc.VectorSubcoreMesh(core_axis_name="c", subcore_axis_name="s")


def _sc_body(x_hbm, srcg_hbm, dstg_hbm, sum_out, cnt_out,
             src_v, dst_v, rows_v, ones_v, acc_sh, cnt_sh, sem):
    cid = lax.axis_index("c")
    sid = lax.axis_index("s")
    wid = sid * NC + cid

    zero16 = jnp.zeros((16,), jnp.float32)

    def _zrows(i, _):
        rows_v[i // 8, pl.ds((i % 8) * 16, 16)] = zero16
        return 0
    lax.fori_loop(0, G * 8, _zrows, 0)

    def _zones(i, _):
        ones_v[i, :] = zero16
        return 0
    lax.fori_loop(0, G, _zones, 0)

    base = sid * RT
    for j in range(RB):
        pltpu.sync_copy(rows_v, acc_sh.at[pl.ds(base + j * G, G)])
        pltpu.sync_copy(ones_v, cnt_sh.at[pl.ds(base + j * G, G)])

    one16 = jnp.ones((16,), jnp.float32)

    def _fones(i, _):
        ones_v[i, :] = one16
        return 0
    lax.fori_loop(0, G, _fones, 0)

    pltpu.sync_copy(srcg_hbm.at[wid], src_v)
    pltpu.sync_copy(dstg_hbm.at[wid], dst_v)

    plsc.subcore_barrier()

    def _edge_group(g, _):
        pltpu.async_copy(x_hbm.at[src_v.at[g]], rows_v, sem).wait()
        pltpu.sync_copy(rows_v, acc_sh.at[dst_v.at[g]], add=True)
        pltpu.sync_copy(ones_v, cnt_sh.at[dst_v.at[g]], add=True)
        return 0
    lax.fori_loop(0, GP, _edge_group, 0)

    plsc.subcore_barrier()

    for j in range(RB):
        off = base + j * G
        pltpu.sync_copy(acc_sh.at[pl.ds(off, G)], sum_out.at[cid, pl.ds(off, G)])
        pltpu.sync_copy(cnt_sh.at[pl.ds(off, G)], cnt_out.at[cid, pl.ds(off, G)])


_sc_aggregate = functools.partial(
    pl.kernel,
    out_type=(
        jax.ShapeDtypeStruct((NC, M, D), jnp.float32),
        jax.ShapeDtypeStruct((NC, M, 16), jnp.float32),
    ),
    mesh=_sc_mesh,
    scratch_types=[
        pltpu.VMEM((GP, G), jnp.int32),
        pltpu.VMEM((GP, G), jnp.int32),
        pltpu.VMEM((G, D), jnp.float32),
        pltpu.VMEM((G, 16), jnp.float32),
        pltpu.VMEM_SHARED((M, D), jnp.float32),
        pltpu.VMEM_SHARED((M, 16), jnp.float32),
        pltpu.SemaphoreType.DMA,
    ],
)(_sc_body)


def _tc_body(s0, s1, c0, c1, xb, wl, wr, bl, ob):
    cnt = c0[:, 0:1] + c1[:, 0:1]
    inv = 1.0 / jnp.maximum(cnt, 1.0)
    agg = (s0[...] + s1[...]) * inv
    out = lax.dot_general(agg, wl[...], (((1,), (1,)), ((), ())),
                          preferred_element_type=jnp.float32)
    out = out + bl[...]
    out = out + lax.dot_general(xb[...], wr[...], (((1,), (1,)), ((), ())),
                                preferred_element_type=jnp.float32)
    nrm = jnp.sqrt(jnp.sum(out * out, axis=1, keepdims=True))
    ob[...] = out / jnp.maximum(nrm, 1e-12)


def _tc_finish(s0, s1, c0, c1, x_pad, W_l, b_l, W_r):
    R = 1024
    grid = (M // R,)
    return pl.pallas_call(
        _tc_body,
        grid=grid,
        in_specs=[
            pl.BlockSpec((R, D), lambda i: (i, 0)),
            pl.BlockSpec((R, D), lambda i: (i, 0)),
            pl.BlockSpec((R, 16), lambda i: (i, 0)),
            pl.BlockSpec((R, 16), lambda i: (i, 0)),
            pl.BlockSpec((R, D), lambda i: (i, 0)),
            pl.BlockSpec((D, D), lambda i: (0, 0)),
            pl.BlockSpec((D, D), lambda i: (0, 0)),
            pl.BlockSpec((1, D), lambda i: (0, 0)),
        ],
        out_specs=pl.BlockSpec((R, D), lambda i: (i, 0)),
        out_shape=jax.ShapeDtypeStruct((M, D), jnp.float32),
    )(s0, s1, c0, c1, x_pad, W_l, W_r, b_l)


def kernel(x, edge_index, W_l, b_l, W_r):
    src = edge_index[0].astype(jnp.int32)
    dst = edge_index[1].astype(jnp.int32)
    pad = E_PAD - N_EDGES
    src = jnp.concatenate([src, jnp.zeros((pad,), jnp.int32)])
    dst = jnp.concatenate([dst, jnp.full((pad,), N_NODES, jnp.int32)])
    srcg = src.reshape(NW, GP, G)
    dstg = dst.reshape(NW, GP, G)
    x_pad = jnp.pad(x, ((0, M - N_NODES), (0, 0)))

    sum_part, cnt_part = _sc_aggregate(x_pad, srcg, dstg)

    out = _tc_finish(sum_part[0], sum_part[1], cnt_part[0], cnt_part[1],
                     x_pad, W_l, b_l, W_r)
    return out[:N_NODES]


# speedup vs baseline: 2.7779x; 2.7779x over previous
import functools

import jax
import jax.numpy as jnp
from jax import lax
from jax.experimental import pallas as pl
from jax.experimental.pallas import tpu as pltpu
from jax.experimental.pallas import tpu_sc as plsc

N_NODES = 10000
N_EDGES = 320000
D = 128

NC = 2
NS = 16
NW = NC * NS
G = 128
GP = -(-N_EDGES // (NW * G))
E_PAD = NW * GP * G
M = 10240
RT = M // NS
RB = RT // G
CW = 16

_sc_mesh = plsc.VectorSubcoreMesh(
    core_axis_name="c", subcore_axis_name="s", num_cores=NC, num_subcores=NS)


def _sum_body(x_hbm, srcg_hbm, dstg_hbm, zrows_hbm, sum_out,
              src_v, dst_v, rows_v, acc_sh, sem):
    cid = lax.axis_index("c")
    sid = lax.axis_index("s")
    wid = sid * NC + cid
    base = sid * RT

    pltpu.sync_copy(zrows_hbm, rows_v)
    for j in range(RB):
        pltpu.sync_copy(rows_v, acc_sh.at[pl.ds(base + j * G, G)])
    plsc.subcore_barrier()

    def _edge_group(g, _):
        pltpu.sync_copy(srcg_hbm.at[wid, g], src_v)
        pltpu.sync_copy(dstg_hbm.at[wid, g], dst_v)
        pltpu.async_copy(x_hbm.at[src_v], rows_v, sem).wait()
        pltpu.sync_copy(rows_v, acc_sh.at[dst_v], add=True)
        return 0
    lax.fori_loop(0, GP, _edge_group, 0)

    plsc.subcore_barrier()
    pltpu.sync_copy(acc_sh.at[pl.ds(base, RT)], sum_out.at[cid, pl.ds(base, RT)])


_sc_sum = functools.partial(
    pl.kernel,
    out_type=jax.ShapeDtypeStruct((NC, M, D), jnp.float32),
    mesh=_sc_mesh,
    scratch_types=[
        pltpu.VMEM((G,), jnp.int32),
        pltpu.VMEM((G,), jnp.int32),
        pltpu.VMEM((G, D), jnp.float32),
        pltpu.VMEM_SHARED((M, D), jnp.float32),
        pltpu.SemaphoreType.DMA,
    ],
)(_sum_body)


def _tc_body(s0, s1, c0, c1, xb, wl, wr, bl, ob):
    cnt = c0[:, 0:1] + c1[:, 0:1]
    inv = 1.0 / jnp.maximum(cnt, 1.0)
    agg = (s0[...] + s1[...]) * inv
    out = lax.dot_general(agg, wl[...], (((1,), (1,)), ((), ())),
                          preferred_element_type=jnp.float32)
    out = out + bl[...]
    out = out + lax.dot_general(xb[...], wr[...], (((1,), (1,)), ((), ())),
                                preferred_element_type=jnp.float32)
    nrm = jnp.sqrt(jnp.sum(out * out, axis=1, keepdims=True))
    ob[...] = out / jnp.maximum(nrm, 1e-12)


def _tc_finish(s0, s1, c0, c1, x_pad, W_l, b_l, W_r):
    R = 1024
    grid = (M // R,)
    return pl.pallas_call(
        _tc_body,
        grid=grid,
        in_specs=[
            pl.BlockSpec((R, D), lambda i: (i, 0)),
            pl.BlockSpec((R, D), lambda i: (i, 0)),
            pl.BlockSpec((R, D), lambda i: (i, 0)),
            pl.BlockSpec((R, D), lambda i: (i, 0)),
            pl.BlockSpec((R, D), lambda i: (i, 0)),
            pl.BlockSpec((D, D), lambda i: (0, 0)),
            pl.BlockSpec((D, D), lambda i: (0, 0)),
            pl.BlockSpec((1, D), lambda i: (0, 0)),
        ],
        out_specs=pl.BlockSpec((R, D), lambda i: (i, 0)),
        out_shape=jax.ShapeDtypeStruct((M, D), jnp.float32),
    )(s0, s1, c0, c1, x_pad, W_l, W_r, b_l.reshape(1, D))


def kernel(x, edge_index, W_l, b_l, W_r):
    src = edge_index[0].astype(jnp.int32)
    dst = edge_index[1].astype(jnp.int32)
    pad = E_PAD - N_EDGES
    src = jnp.concatenate([src, jnp.zeros((pad,), jnp.int32)])
    dst = jnp.concatenate([dst, jnp.full((pad,), N_NODES, jnp.int32)])
    srcg = src.reshape(NW, GP, G)
    dstg = dst.reshape(NW, GP, G)
    x_pad = jnp.pad(x, ((0, M - N_NODES), (0, 0)))
    zrows = jnp.zeros((G, D), jnp.float32)
    ones_tbl = jnp.ones((M, D), jnp.float32)

    sum_part = _sc_sum(x_pad, srcg, dstg, zrows)
    cnt_part = _sc_sum(ones_tbl, srcg, dstg, zrows)

    out = _tc_finish(sum_part[0], sum_part[1], cnt_part[0], cnt_part[1],
                     x_pad, W_l, b_l, W_r)
    return out[:N_NODES]
